# baseline (speedup 1.0000x reference)
import math

import jax
import jax.numpy as jnp
from jax import lax
from jax.experimental import pallas as pl
from jax.experimental.pallas import tpu as pltpu
from jax.experimental.pallas import tpu_sc as plsc

D_MODEL = 64
SCALE = math.sqrt(D_MODEL)

NUM_CORES = 2
NUM_SUBCORES = 16
NUM_WORKERS = NUM_CORES * NUM_SUBCORES

VOCAB = 1000000
LANES = 128
N_TCOL = (VOCAB + LANES - 1) // LANES
TCOL_PER_WORKER = (N_TCOL + NUM_WORKERS - 1) // NUM_WORKERS
N_PAIR = N_TCOL * (LANES // 2)

B_TOTAL = 4096 * 50
ROWS_PER_WORKER = B_TOTAL // NUM_WORKERS
CHUNK = 256
NUM_CHUNKS = ROWS_PER_WORKER // CHUNK
SUBGATHER = 128
NUM_SUB = CHUNK // SUBGATHER
ROW_UNROLL = 4
D_PAIR = 2 * D_MODEL


def _relayout_kernel(lut_t, tail_t, out_hbm,
                     buf0, buf1, pair0, pair1, rsem0, rsem1, wsem0, wsem1):
    wid = lax.axis_index("s") * NUM_CORES + lax.axis_index("c")
    tc0 = wid * TCOL_PER_WORKER
    n_my = jnp.minimum(TCOL_PER_WORKER, jnp.maximum(N_TCOL - tc0, 0))
    bufs = (buf0, buf1)
    pairs = (pair0, pair1)
    rsems = (rsem0, rsem1)
    wsems = (wsem0, wsem1)

    iotas = [lax.iota(jnp.int32, 16) + 16 * kk for kk in range(4)]

    def fire(t, b):
        tc = tc0 + t

        @pl.when(tc < N_TCOL - 1)
        def _():
            for tr in range(8):
                pltpu.async_copy(
                    lut_t.at[pl.ds(8 * tr, 8), pl.ds(tc * LANES, LANES)],
                    bufs[b].at[pl.ds(8 * tr, 8), :],
                    rsems[b],
                )

        @pl.when(tc == N_TCOL - 1)
        def _():
            for tr in range(8):
                pltpu.async_copy(
                    tail_t.at[pl.ds(8 * tr, 8), :],
                    bufs[b].at[pl.ds(8 * tr, 8), :],
                    rsems[b],
                )

    def drain(b):
        for tr in range(8):
            pltpu.make_async_copy(
                lut_t.at[pl.ds(0, 8), pl.ds(0, LANES)],
                bufs[b].at[pl.ds(8 * tr, 8), :],
                rsems[b],
            ).wait()

    def step(t, b):
        @pl.when(t + 1 < n_my)
        def _():
            fire(t + 1, (b + 1) % 2)

        drain(b)
        tc = tc0 + t
        buf = bufs[b]
        pair = pairs[b]

        @pl.when(t >= 2)
        def _():
            pltpu.make_async_copy(
                pair, out_hbm.at[pl.ds(0, 64)], wsems[b]).wait()

        def shuffle(p, _):
            for k in range(8):
                col = jnp.full((16,), 2 * p + (1 if k >= 4 else 0),
                               dtype=jnp.int32)
                vals = plsc.load_gather(buf, [iotas[k % 4], col])
                pair[p, pl.ds(16 * k, 16)] = vals
            return None
        lax.fori_loop(0, 64, shuffle, None)

        pltpu.async_copy(pair, out_hbm.at[pl.ds(tc * 64, 64)], wsems[b])

    def super_body(t2, _):
        for b in range(2):
            t = 2 * t2 + b

            @pl.when(t < n_my)
            def _(t=t, b=b):
                step(t, b)
        return None

    @pl.when(n_my > 0)
    def _():
        fire(0, 0)
        lax.fori_loop(0, (TCOL_PER_WORKER + 1) // 2, super_body, None)
        for b in range(2):
            @pl.when(n_my > b)
            def _(b=b):
                pltpu.make_async_copy(
                    pairs[b], out_hbm.at[pl.ds(0, 64)], wsems[b]).wait()


def _gather_kernel(lut_hbm, idx_hbm, out_hbm,
                   idx0, idx1, rows0, rows1, gsem0, gsem1, wsem0, wsem1):
    wid = lax.axis_index("s") * NUM_CORES + lax.axis_index("c")
    base = wid * ROWS_PER_WORKER
    idx_bufs = (idx0, idx1)
    row_bufs = (rows0, rows1)
    gsems = (gsem0, gsem1)
    wsems = (wsem0, wsem1)

    def fire(c):
        b = c % 2
        row0 = base + c * CHUNK
        pltpu.sync_copy(idx_hbm.at[pl.ds(row0, CHUNK)], idx_bufs[b])
        cps = []
        for j in range(NUM_SUB):
            cps.append(pltpu.async_copy(
                lut_hbm.at[idx_bufs[b].at[pl.ds(j * SUBGATHER, SUBGATHER)]],
                row_bufs[b].at[pl.ds(j * SUBGATHER, SUBGATHER)],
                gsems[b],
            ))
        return cps

    def write(c):
        b = c % 2
        row0 = base + c * CHUNK
        return pltpu.async_copy(row_bufs[b], out_hbm.at[pl.ds(row0, CHUNK)],
                                wsems[b])

    pending_g = {0: fire(0)}
    pending_w = {}
    for c in range(NUM_CHUNKS):
        b = c % 2
        if c + 1 < NUM_CHUNKS:
            if c - 1 in pending_w:
                pending_w.pop(c - 1).wait()
            pending_g[c + 1] = fire(c + 1)
        for cp in pending_g.pop(c):
            cp.wait()

        rows = row_bufs[b]

        def mul_body(i, _):
            for rr in range(ROW_UNROLL):
                r = i * ROW_UNROLL + rr
                for k in range(D_PAIR // 16):
                    sl = pl.ds(k * 16, 16)
                    rows[r, sl] = rows[r, sl] * SCALE
            return None
        lax.fori_loop(0, CHUNK // ROW_UNROLL, mul_body, None)

        pending_w[c] = write(c)
    for cp in pending_w.values():
        cp.wait()


@jax.jit
def kernel(x, lut):
    idx = x.reshape(-1).astype(jnp.int32)
    ih = idx >> 1
    parity = (idx & 1).astype(jnp.int32)
    lut_t = lut.T
    tail_t = jnp.pad(lut_t[:, (N_TCOL - 1) * LANES:],
                     ((0, 0), (0, N_TCOL * LANES - VOCAB)))
    mesh = plsc.VectorSubcoreMesh(core_axis_name="c", subcore_axis_name="s")

    lut2 = pl.kernel(
        _relayout_kernel,
        mesh=mesh,
        compiler_params=pltpu.CompilerParams(use_tc_tiling_on_sc=True,
                                             needs_layout_passes=False),
        out_type=jax.ShapeDtypeStruct((N_PAIR, D_PAIR), jnp.float32),
        scratch_types=[
            pltpu.VMEM((64, LANES), jnp.float32),
            pltpu.VMEM((64, LANES), jnp.float32),
            pltpu.VMEM((64, D_PAIR), jnp.float32),
            pltpu.VMEM((64, D_PAIR), jnp.float32),
            pltpu.SemaphoreType.DMA,
            pltpu.SemaphoreType.DMA,
            pltpu.SemaphoreType.DMA,
            pltpu.SemaphoreType.DMA,
        ],
    )(lut_t, tail_t)

    out = pl.kernel(
        _gather_kernel,
        mesh=mesh,
        compiler_params=pltpu.CompilerParams(use_tc_tiling_on_sc=True),
        out_type=jax.ShapeDtypeStruct((B_TOTAL, D_PAIR), jnp.float32),
        scratch_types=[
            pltpu.VMEM((CHUNK,), jnp.int32),
            pltpu.VMEM((CHUNK,), jnp.int32),
            pltpu.VMEM((CHUNK, D_PAIR), jnp.float32),
            pltpu.VMEM((CHUNK, D_PAIR), jnp.float32),
            pltpu.SemaphoreType.DMA,
            pltpu.SemaphoreType.DMA,
            pltpu.SemaphoreType.DMA,
            pltpu.SemaphoreType.DMA,
        ],
    )(lut2, ih)
    sel = jnp.where((parity == 1)[:, None], out[:, D_MODEL:], out[:, :D_MODEL])
    return sel.reshape(x.shape[0], x.shape[1], D_MODEL)


# speedup vs baseline: 1.0006x; 1.0006x over previous
import math

import jax
import jax.numpy as jnp
from jax import lax
from jax.experimental import pallas as pl
from jax.experimental.pallas import tpu as pltpu
from jax.experimental.pallas import tpu_sc as plsc

D_MODEL = 64
SCALE = math.sqrt(D_MODEL)

NUM_CORES = 2
NUM_SUBCORES = 16
NUM_WORKERS = NUM_CORES * NUM_SUBCORES

VOCAB = 1000000
LANES = 128
N_TCOL = (VOCAB + LANES - 1) // LANES
TCOL_PER_WORKER = (N_TCOL + NUM_WORKERS - 1) // NUM_WORKERS
N_PAIR = N_TCOL * (LANES // 2)

B_TOTAL = 4096 * 50
ROWS_PER_WORKER = B_TOTAL // NUM_WORKERS
CHUNK = 256
NUM_CHUNKS = ROWS_PER_WORKER // CHUNK
SUBGATHER = 128
NUM_SUB = CHUNK // SUBGATHER
ROW_UNROLL = 4
D_PAIR = 2 * D_MODEL


def _relayout_kernel(lut_t, tail_t, out_hbm,
                     buf0, buf1, pair0, pair1, rsem0, rsem1, wsem0, wsem1):
    wid = lax.axis_index("s") * NUM_CORES + lax.axis_index("c")
    tc0 = wid * TCOL_PER_WORKER
    n_my = jnp.minimum(TCOL_PER_WORKER, jnp.maximum(N_TCOL - tc0, 0))
    bufs = (buf0, buf1)
    pairs = (pair0, pair1)
    rsems = (rsem0, rsem1)
    wsems = (wsem0, wsem1)

    iotas = [lax.iota(jnp.int32, 16) + 16 * kk for kk in range(4)]

    def fire(t, b):
        tc = tc0 + t

        @pl.when(tc < N_TCOL - 1)
        def _():
            for tr in range(8):
                pltpu.async_copy(
                    lut_t.at[pl.ds(8 * tr, 8), pl.ds(tc * LANES, LANES)],
                    bufs[b].at[pl.ds(8 * tr, 8), :],
                    rsems[b],
                )

        @pl.when(tc == N_TCOL - 1)
        def _():
            for tr in range(8):
                pltpu.async_copy(
                    tail_t.at[pl.ds(8 * tr, 8), :],
                    bufs[b].at[pl.ds(8 * tr, 8), :],
                    rsems[b],
                )

    def drain(b):
        for tr in range(8):
            pltpu.make_async_copy(
                lut_t.at[pl.ds(0, 8), pl.ds(0, LANES)],
                bufs[b].at[pl.ds(8 * tr, 8), :],
                rsems[b],
            ).wait()

    def step(t, b):
        @pl.when(t + 1 < n_my)
        def _():
            fire(t + 1, (b + 1) % 2)

        drain(b)
        tc = tc0 + t
        buf = bufs[b]
        pair = pairs[b]

        @pl.when(t >= 2)
        def _():
            pltpu.make_async_copy(
                pair, out_hbm.at[pl.ds(0, 64)], wsems[b]).wait()

        def shuffle(i, _):
            for pp in range(4):
                p = i * 4 + pp
                cols = (jnp.full((16,), 2 * p, dtype=jnp.int32),
                        jnp.full((16,), 2 * p + 1, dtype=jnp.int32))
                for k in range(8):
                    vals = plsc.load_gather(buf, [iotas[k % 4], cols[k // 4]])
                    pair[p, pl.ds(16 * k, 16)] = vals
            return None
        lax.fori_loop(0, 16, shuffle, None)

        pltpu.async_copy(pair, out_hbm.at[pl.ds(tc * 64, 64)], wsems[b])

    def super_body(t2, _):
        for b in range(2):
            t = 2 * t2 + b

            @pl.when(t < n_my)
            def _(t=t, b=b):
                step(t, b)
        return None

    @pl.when(n_my > 0)
    def _():
        fire(0, 0)
        lax.fori_loop(0, (TCOL_PER_WORKER + 1) // 2, super_body, None)
        for b in range(2):
            @pl.when(n_my > b)
            def _(b=b):
                pltpu.make_async_copy(
                    pairs[b], out_hbm.at[pl.ds(0, 64)], wsems[b]).wait()


def _gather_kernel(lut_hbm, idx_hbm, out_hbm,
                   idx0, idx1, rows0, rows1, gsem0, gsem1, wsem0, wsem1):
    wid = lax.axis_index("s") * NUM_CORES + lax.axis_index("c")
    base = wid * ROWS_PER_WORKER
    idx_bufs = (idx0, idx1)
    row_bufs = (rows0, rows1)
    gsems = (gsem0, gsem1)
    wsems = (wsem0, wsem1)

    def fire(c):
        b = c % 2
        row0 = base + c * CHUNK
        pltpu.sync_copy(idx_hbm.at[pl.ds(row0, CHUNK)], idx_bufs[b])
        cps = []
        for j in range(NUM_SUB):
            cps.append(pltpu.async_copy(
                lut_hbm.at[idx_bufs[b].at[pl.ds(j * SUBGATHER, SUBGATHER)]],
                row_bufs[b].at[pl.ds(j * SUBGATHER, SUBGATHER)],
                gsems[b],
            ))
        return cps

    def write(c):
        b = c % 2
        row0 = base + c * CHUNK
        return pltpu.async_copy(row_bufs[b], out_hbm.at[pl.ds(row0, CHUNK)],
                                wsems[b])

    pending_g = {0: fire(0)}
    pending_w = {}
    for c in range(NUM_CHUNKS):
        b = c % 2
        if c + 1 < NUM_CHUNKS:
            if c - 1 in pending_w:
                pending_w.pop(c - 1).wait()
            pending_g[c + 1] = fire(c + 1)
        for cp in pending_g.pop(c):
            cp.wait()

        rows = row_bufs[b]

        def mul_body(i, _):
            for rr in range(ROW_UNROLL):
                r = i * ROW_UNROLL + rr
                for k in range(D_PAIR // 16):
                    sl = pl.ds(k * 16, 16)
                    rows[r, sl] = rows[r, sl] * SCALE
            return None
        lax.fori_loop(0, CHUNK // ROW_UNROLL, mul_body, None)

        pending_w[c] = write(c)
    for cp in pending_w.values():
        cp.wait()


@jax.jit
def kernel(x, lut):
    idx = x.reshape(-1).astype(jnp.int32)
    ih = idx >> 1
    parity = (idx & 1).astype(jnp.int32)
    lut_t = lut.T
    tail_t = jnp.pad(lut_t[:, (N_TCOL - 1) * LANES:],
                     ((0, 0), (0, N_TCOL * LANES - VOCAB)))
    mesh = plsc.VectorSubcoreMesh(core_axis_name="c", subcore_axis_name="s")

    lut2 = pl.kernel(
        _relayout_kernel,
        mesh=mesh,
        compiler_params=pltpu.CompilerParams(use_tc_tiling_on_sc=True,
                                             needs_layout_passes=False,
                                             disable_bounds_checks=True),
        out_type=jax.ShapeDtypeStruct((N_PAIR, D_PAIR), jnp.float32),
        scratch_types=[
            pltpu.VMEM((64, LANES), jnp.float32),
            pltpu.VMEM((64, LANES), jnp.float32),
            pltpu.VMEM((64, D_PAIR), jnp.float32),
            pltpu.VMEM((64, D_PAIR), jnp.float32),
            pltpu.SemaphoreType.DMA,
            pltpu.SemaphoreType.DMA,
            pltpu.SemaphoreType.DMA,
            pltpu.SemaphoreType.DMA,
        ],
    )(lut_t, tail_t)

    out = pl.kernel(
        _gather_kernel,
        mesh=mesh,
        compiler_params=pltpu.CompilerParams(use_tc_tiling_on_sc=True),
        out_type=jax.ShapeDtypeStruct((B_TOTAL, D_PAIR), jnp.float32),
        scratch_types=[
            pltpu.VMEM((CHUNK,), jnp.int32),
            pltpu.VMEM((CHUNK,), jnp.int32),
            pltpu.VMEM((CHUNK, D_PAIR), jnp.float32),
            pltpu.VMEM((CHUNK, D_PAIR), jnp.float32),
            pltpu.SemaphoreType.DMA,
            pltpu.SemaphoreType.DMA,
            pltpu.SemaphoreType.DMA,
            pltpu.SemaphoreType.DMA,
        ],
    )(lut2, ih)
    sel = jnp.where((parity == 1)[:, None], out[:, D_MODEL:], out[:, :D_MODEL])
    return sel.reshape(x.shape[0], x.shape[1], D_MODEL)
